# baseline (speedup 1.0000x reference)
import jax
import jax.numpy as jnp
from jax.experimental import pallas as pl
from jax.experimental.pallas import tpu as pltpu

_COLS = 8192
_NB = 16


def _embed_block(gidx_ref, fta_ref, ftb_ref, mask_ref, wt_ref, bias_ref,
                 cls_ref, out_ref):
    i = pl.program_id(0)
    kh = fta_ref.shape[0]
    lin = jax.lax.dot_general(
        wt_ref[:kh, :], fta_ref[...],
        dimension_numbers=(((0,), (0,)), ((), ())),
        preferred_element_type=jnp.float32,
    ) + jax.lax.dot_general(
        wt_ref[kh:, :], ftb_ref[...],
        dimension_numbers=(((0,), (0,)), ((), ())),
        preferred_element_type=jnp.float32,
    )
    bias_col = bias_ref[...].reshape(bias_ref.shape[1], 1)
    masked = jnp.where(mask_ref[...], lin + bias_col, 0.0)
    out_ref[...] = masked.T
    base = i * _COLS
    for k in range(_NB):
        g = gidx_ref[k]
        local = g - base

        @pl.when((local >= 0) & (local < _COLS))
        def _():
            out_ref[pl.ds(local, 1), :] = cls_ref[...]


def kernel(feat, amask, g_idx, b_idx, W, bias, cls_token):
    n, token_dim = feat.shape
    emb_dim = W.shape[0]
    ft = feat.T
    wt = W.T
    maskb = amask.reshape(1, n)
    grid_spec = pltpu.PrefetchScalarGridSpec(
        num_scalar_prefetch=1,
        grid=(n // _COLS,),
        in_specs=[
            pl.BlockSpec((token_dim // 2, _COLS), lambda i, g: (0, i)),
            pl.BlockSpec((token_dim // 2, _COLS), lambda i, g: (1, i)),
            pl.BlockSpec((1, _COLS), lambda i, g: (0, i)),
            pl.BlockSpec((token_dim, emb_dim), lambda i, g: (0, 0)),
            pl.BlockSpec((1, emb_dim), lambda i, g: (0, 0)),
            pl.BlockSpec((1, emb_dim), lambda i, g: (0, 0)),
        ],
        out_specs=pl.BlockSpec((_COLS, emb_dim), lambda i, g: (i, 0)),
    )
    out = pl.pallas_call(
        _embed_block,
        grid_spec=grid_spec,
        out_shape=jax.ShapeDtypeStruct((n, emb_dim), jnp.float32),
        compiler_params=pltpu.CompilerParams(
            dimension_semantics=("arbitrary",),
        ),
    )(
        g_idx.astype(jnp.int32),
        ft,
        ft,
        maskb,
        wt,
        bias.reshape(1, emb_dim),
        cls_token.reshape(1, emb_dim),
    )
    return (out, amask, g_idx, b_idx)


# speedup vs baseline: 1.0573x; 1.0573x over previous
import jax
import jax.numpy as jnp
from jax.experimental import pallas as pl
from jax.experimental.pallas import tpu as pltpu

_COLS = 8192
_NB = 16


def _embed_block(gidx_ref, ft_ref, mask_ref, wt_ref, bias_ref, cls_ref, out_ref):
    i = pl.program_id(0)
    t = mask_ref.shape[1]
    lin = jax.lax.dot_general(
        wt_ref[...], ft_ref[...],
        dimension_numbers=(((0,), (0,)), ((), ())),
        preferred_element_type=jnp.float32,
    )
    biased = lin + bias_ref[...].reshape(bias_ref.shape[1], 1)
    rows = _COLS // t
    parts = [
        jnp.where(mask_ref[pl.ds(rows * i + r, 1), :],
                  biased[:, r * t:(r + 1) * t], 0.0)
        for r in range(rows)
    ]
    masked = jnp.concatenate(parts, axis=1) if rows > 1 else parts[0]
    out_ref[...] = masked.T
    base = i * _COLS
    for k in range(_NB):
        g = gidx_ref[k]
        local = g - base

        @pl.when((local >= 0) & (local < _COLS))
        def _():
            out_ref[pl.ds(local, 1), :] = cls_ref[...]


def kernel(feat, amask, g_idx, b_idx, W, bias, cls_token):
    n, token_dim = feat.shape
    emb_dim = W.shape[0]
    nb_rows, t = amask.shape
    ft = feat.T
    wt = W.T
    grid_spec = pltpu.PrefetchScalarGridSpec(
        num_scalar_prefetch=1,
        grid=(n // _COLS,),
        in_specs=[
            pl.BlockSpec((token_dim, _COLS), lambda i, g: (0, i)),
            pl.BlockSpec((nb_rows, t), lambda i, g: (0, 0)),
            pl.BlockSpec((token_dim, emb_dim), lambda i, g: (0, 0)),
            pl.BlockSpec((1, emb_dim), lambda i, g: (0, 0)),
            pl.BlockSpec((1, emb_dim), lambda i, g: (0, 0)),
        ],
        out_specs=pl.BlockSpec((_COLS, emb_dim), lambda i, g: (i, 0)),
    )
    out = pl.pallas_call(
        _embed_block,
        grid_spec=grid_spec,
        out_shape=jax.ShapeDtypeStruct((n, emb_dim), jnp.float32),
        compiler_params=pltpu.CompilerParams(
            dimension_semantics=("arbitrary",),
        ),
    )(
        g_idx.astype(jnp.int32),
        ft,
        amask,
        wt,
        bias.reshape(1, emb_dim),
        cls_token.reshape(1, emb_dim),
    )
    return (out, amask, g_idx, b_idx)
